# baseline (speedup 1.0000x reference)
import functools

import numpy as np
import jax
import jax.numpy as jnp
from jax import lax
from jax.experimental import pallas as pl
from jax.experimental.pallas import tpu as pltpu
from jax.experimental.pallas import tpu_sc as plsc

NUM_FIELDS = 26
FIELD_SIZE = 38461
EMBED_DIM = 32
BATCH = 4096
NUM_IX = NUM_FIELDS * (NUM_FIELDS - 1) // 2
EMB_FLAT = NUM_FIELDS * EMBED_DIM

_CHUNK = 128


def _sc_gather(table, idx3):
    info = plsc.get_sparse_core_info()
    nc, ns = info.num_cores, info.num_subcores
    nw = nc * ns
    n_chunks = idx3.shape[1]
    b_per_w = n_chunks * _CHUNK
    n = nw * b_per_w
    mesh = plsc.VectorSubcoreMesh(core_axis_name="c", subcore_axis_name="s")

    @functools.partial(
        pl.kernel,
        mesh=mesh,
        compiler_params=pltpu.CompilerParams(use_tc_tiling_on_sc=False),
        out_type=jax.ShapeDtypeStruct((n, EMBED_DIM), jnp.float32),
        scratch_types=[
            pltpu.VMEM((n_chunks, _CHUNK), jnp.int32),
            pltpu.VMEM((b_per_w, EMBED_DIM), jnp.float32),
            pltpu.SemaphoreType.DMA,
        ],
    )
    def gather_kernel(table_hbm, idx_hbm, out_hbm, idx_v, rows_v, sem):
        wid = lax.axis_index("s") * nc + lax.axis_index("c")
        base = wid * b_per_w
        pltpu.sync_copy(idx_hbm.at[wid], idx_v)
        copies = [
            pltpu.make_async_copy(
                table_hbm.at[idx_v.at[j]],
                rows_v.at[pl.ds(j * _CHUNK, _CHUNK)],
                sem,
            )
            for j in range(n_chunks)
        ]
        for c in copies:
            c.start()
        for c in copies:
            c.wait()
        pltpu.sync_copy(rows_v, out_hbm.at[pl.ds(base, b_per_w)])

    return gather_kernel(table, idx3)


def _mlp_body(e_ref, w1a_ref, w1b_ref, b1_ref, w2_ref, b2_ref, w3_ref, b3_ref,
              o_ref):
    e = e_ref[...]
    tb = e.shape[0]
    e3 = e.reshape(tb, NUM_FIELDS, EMBED_DIM)
    parts = []
    for i in range(NUM_FIELDS - 1):
        parts.append(jnp.sum(e3[:, i + 1:, :] * e3[:, i:i + 1, :], axis=2))
    prod = jnp.concatenate(parts, axis=1)
    h = jnp.dot(e, w1a_ref[...], preferred_element_type=jnp.float32)
    h = h + jnp.dot(prod, w1b_ref[...], preferred_element_type=jnp.float32)
    h = jnp.maximum(h + b1_ref[...], 0.0)
    h = jnp.dot(h, w2_ref[...], preferred_element_type=jnp.float32)
    h = jnp.maximum(h + b2_ref[...], 0.0)
    o = jnp.sum(h * w3_ref[...], axis=1) + b3_ref[0, 0]
    o_ref[...] = jax.nn.sigmoid(o)[None, None, :]


def _mlp_call(emb, w1a, w1b, b1f, w2f, b2f, w3r, b3s, tb=512):
    grid = (BATCH // tb,)
    const = lambda i: (0, 0)
    out = pl.pallas_call(
        _mlp_body,
        grid=grid,
        in_specs=[
            pl.BlockSpec((tb, EMB_FLAT), lambda i: (i, 0)),
            pl.BlockSpec((EMB_FLAT, 256), const),
            pl.BlockSpec((NUM_IX, 256), const),
            pl.BlockSpec((1, 256), const),
            pl.BlockSpec((256, 128), const),
            pl.BlockSpec((1, 128), const),
            pl.BlockSpec((1, 128), const),
            pl.BlockSpec((1, 1), const),
        ],
        out_specs=pl.BlockSpec((1, 1, tb), lambda i: (i, 0, 0)),
        out_shape=jax.ShapeDtypeStruct((BATCH // tb, 1, tb), jnp.float32),
    )(emb, w1a, w1b, b1f, w2f, b2f, w3r, b3s)
    return out.reshape(BATCH)


def kernel(x, table, W1, b1, g1, be1, W2, b2, g2, be2, W3, b3):
    offsets = (np.arange(NUM_FIELDS) * FIELD_SIZE).astype(np.int32)
    idx = (x.astype(jnp.int32) + offsets[None, :]).reshape(-1)
    info = plsc.get_sparse_core_info()
    nw = info.num_cores * info.num_subcores
    idx3 = idx.reshape(nw, idx.shape[0] // (nw * _CHUNK), _CHUNK)

    rows = _sc_gather(table, idx3)
    emb = rows.reshape(BATCH, EMB_FLAT)

    c = np.float32(1.0 / np.sqrt(1.0 + 1e-5))
    s1 = g1 * c
    w1f = W1 * s1[None, :]
    b1f = (b1 * s1 + be1)[None, :]
    s2 = g2 * c
    w2f = W2 * s2[None, :]
    b2f = (b2 * s2 + be2)[None, :]
    w3r = W3.reshape(1, -1)
    b3s = b3.reshape(1, 1)

    return _mlp_call(emb, w1f[:EMB_FLAT], w1f[EMB_FLAT:], b1f, w2f, b2f,
                     w3r, b3s)


# speedup vs baseline: 1.7090x; 1.7090x over previous
import functools

import numpy as np
import jax
import jax.numpy as jnp
from jax import lax
from jax.experimental import pallas as pl
from jax.experimental.pallas import tpu as pltpu
from jax.experimental.pallas import tpu_sc as plsc

NUM_FIELDS = 26
FIELD_SIZE = 38461
EMBED_DIM = 32
BATCH = 4096
NUM_IX = NUM_FIELDS * (NUM_FIELDS - 1) // 2
EMB_FLAT = NUM_FIELDS * EMBED_DIM

_CHUNK = 128


def _sc_gather(table, idx3):
    info = plsc.get_sparse_core_info()
    nc, ns = info.num_cores, info.num_subcores
    nw = nc * ns
    n_chunks = idx3.shape[1]
    rows_per_w = n_chunks * _CHUNK
    mesh = plsc.VectorSubcoreMesh(core_axis_name="c", subcore_axis_name="s")

    @functools.partial(
        pl.kernel,
        mesh=mesh,
        compiler_params=pltpu.CompilerParams(use_tc_tiling_on_sc=False),
        out_type=jax.ShapeDtypeStruct((NUM_FIELDS, BATCH, 128), jnp.float32),
        scratch_types=[
            pltpu.VMEM((n_chunks, _CHUNK), jnp.int32),
            pltpu.VMEM((rows_per_w, EMBED_DIM), jnp.float32),
            pltpu.SemaphoreType.DMA,
            pltpu.SemaphoreType.DMA,
        ],
    )
    def gather_kernel(tab_hbm, idx_hbm, out_hbm, idx_v, rows_v, sem, sem2):
        wid = lax.axis_index("s") * nc + lax.axis_index("c")
        base = wid * rows_per_w
        pltpu.sync_copy(idx_hbm.at[wid], idx_v)
        gathers = [
            pltpu.make_async_copy(
                tab_hbm.at[idx_v.at[c]],
                rows_v.at[pl.ds(c * _CHUNK, _CHUNK)],
                sem,
            )
            for c in range(n_chunks)
        ]
        for g in gathers:
            g.start()
        for g in gathers:
            g.wait()
        writes = []
        for c in range(n_chunks):
            r0 = base + c * _CHUNK
            f = r0 // BATCH
            b0 = r0 % BATCH
            writes.append(pltpu.make_async_copy(
                rows_v.at[pl.ds(c * _CHUNK, _CHUNK)],
                out_hbm.at[f, pl.ds(b0, _CHUNK), pl.ds(0, EMBED_DIM)],
                sem2,
            ))
        for wcp in writes:
            wcp.start()
        for wcp in writes:
            wcp.wait()

    return gather_kernel(table, idx3)


def _mlp_body(e_ref, w1a_ref, w1b_ref, b1_ref, w2_ref, b2_ref, w3_ref, b3_ref,
              o_ref):
    v = e_ref[...]
    tb = v.shape[1]
    et = jnp.concatenate(
        [jnp.transpose(v[f])[:EMBED_DIM, :] for f in range(NUM_FIELDS)],
        axis=0)
    h = jnp.dot(w1a_ref[...], et, preferred_element_type=jnp.float32)
    off = 0
    for dlt in range(1, NUM_FIELDS):
        k = NUM_FIELDS - dlt
        rows = k * EMBED_DIM
        a = et[:rows, :] * et[dlt * EMBED_DIM:, :]
        p = jnp.sum(a.reshape(k, EMBED_DIM, tb), axis=1)
        h = h + jnp.dot(w1b_ref[:, off:off + k], p,
                        preferred_element_type=jnp.float32)
        off += k
    h = jnp.maximum(h + b1_ref[...], 0.0)
    h = jnp.dot(w2_ref[...], h, preferred_element_type=jnp.float32)
    h = jnp.maximum(h + b2_ref[...], 0.0)
    o = jnp.sum(h * w3_ref[...], axis=0) + b3_ref[0, 0]
    o_ref[...] = jax.nn.sigmoid(o)[None, None, :]


def _mlp_call(emb3, w1a, w1b, b1f, w2f, b2f, w3c, b3s, tb=512):
    grid = (BATCH // tb,)
    const = lambda i: (0, 0)
    out = pl.pallas_call(
        _mlp_body,
        grid=grid,
        in_specs=[
            pl.BlockSpec((NUM_FIELDS, tb, 128), lambda i: (0, i, 0)),
            pl.BlockSpec((256, EMB_FLAT), const),
            pl.BlockSpec((256, NUM_IX), const),
            pl.BlockSpec((256, 1), const),
            pl.BlockSpec((128, 256), const),
            pl.BlockSpec((128, 1), const),
            pl.BlockSpec((128, 1), const),
            pl.BlockSpec((1, 1), const),
        ],
        out_specs=pl.BlockSpec((1, 1, tb), lambda i: (i, 0, 0)),
        out_shape=jax.ShapeDtypeStruct((BATCH // tb, 1, tb), jnp.float32),
    )(emb3, w1a, w1b, b1f, w2f, b2f, w3c, b3s)
    return out.reshape(BATCH)


def _delta_perm():
    row, col = np.triu_indices(NUM_FIELDS, k=1)
    lut = {(i, j): n for n, (i, j) in enumerate(zip(row, col))}
    perm = [lut[(f, f + dlt)]
            for dlt in range(1, NUM_FIELDS)
            for f in range(NUM_FIELDS - dlt)]
    return np.asarray(perm, dtype=np.int32)


_PERM = _delta_perm()


def kernel(x, table, W1, b1, g1, be1, W2, b2, g2, be2, W3, b3):
    offsets = (np.arange(NUM_FIELDS) * FIELD_SIZE).astype(np.int32)
    idx = x.astype(jnp.int32).T + offsets[:, None]
    idx3 = idx.reshape(32, NUM_FIELDS, _CHUNK)

    emb3 = _sc_gather(table, idx3)

    c = np.float32(1.0 / np.sqrt(1.0 + 1e-5))
    w1f = (W1 * (g1 * c)[None, :]).T
    b1f = ((b1 * g1 * c) + be1)[:, None]
    w1a = w1f[:, :EMB_FLAT]
    w1b = w1f[:, EMB_FLAT:][:, _PERM]
    w2f = (W2 * (g2 * c)[None, :]).T
    b2f = ((b2 * g2 * c) + be2)[:, None]
    b3s = b3.reshape(1, 1)

    return _mlp_call(emb3, w1a, w1b, b1f, w2f, b2f, W3, b3s)
